# baseline (speedup 1.0000x reference)
import functools

import jax
import jax.numpy as jnp
from jax import lax
from jax.experimental import pallas as pl
from jax.experimental.pallas import tpu as pltpu
from jax.experimental.pallas import tpu_sc as plsc

D = 64
NB = 16384
M = 8
K = 8
NC = 2
NS = 16
NW = NC * NS
BPW = NB // NW
CHUNK = 64
NCHUNKS = BPW // CHUNK
GROUPS = CHUNK // 16
XSTREAMS = CHUNK * M // 128

_mesh = plsc.VectorSubcoreMesh(core_axis_name="c", subcore_axis_name="s")


@functools.partial(
    pl.kernel,
    out_type=(
        jax.ShapeDtypeStruct((NB * M,), jnp.float32),
        jax.ShapeDtypeStruct((NB, D), jnp.float32),
        jax.ShapeDtypeStruct((K, D), jnp.float32),
    ),
    mesh=_mesh,
    compiler_params=pltpu.CompilerParams(needs_layout_passes=False,
                                         use_tc_tiling_on_sc=False),
    scratch_types=[
        pltpu.VMEM((CHUNK,), jnp.int32),
        pltpu.VMEM((XSTREAMS, 128), jnp.int32),
        pltpu.VMEM((CHUNK, D), jnp.float32),
        pltpu.VMEM((CHUNK * M, D), jnp.float32),
        pltpu.VMEM((CHUNK * M,), jnp.float32),
        pltpu.VMEM((K,), jnp.int32),
        pltpu.VMEM((K, D), jnp.float32),
        pltpu.SemaphoreType.DMA,
    ],
)
def _sc_scores(cen_hbm, ctx2d_hbm, table_hbm, neg_hbm,
               pos_hbm, cemb_hbm, nemb_hbm,
               cidx_v, xidx_v, crows_v, xrows_v, pos_v, nidx_v, nrows_v, sem):
    wid = lax.axis_index("s") * NC + lax.axis_index("c")
    iota = lax.iota(jnp.int32, 16)

    @pl.when(wid == 0)
    def _():
        pltpu.sync_copy(neg_hbm, nidx_v)
        pltpu.async_copy(table_hbm.at[nidx_v], nrows_v, sem).wait()
        pltpu.sync_copy(nrows_v, nemb_hbm)

    def chunk_body(c, carry):
        base = wid * BPW + c * CHUNK
        pltpu.sync_copy(cen_hbm.at[pl.ds(base, CHUNK)], cidx_v)
        for j in range(XSTREAMS):
            pltpu.sync_copy(ctx2d_hbm.at[pl.ds(base * M + j * 128, 128)],
                            xidx_v.at[j])
        cp = pltpu.async_copy(table_hbm.at[cidx_v], crows_v, sem)
        xps = [
            pltpu.async_copy(table_hbm.at[xidx_v.at[j]],
                             xrows_v.at[pl.ds(j * 128, 128)], sem)
            for j in range(XSTREAMS)
        ]
        cp.wait()
        for x in xps:
            x.wait()

        def group_body(g, carry2):
            bidx = g * 16 + iota
            rowv = [g * 128 + m + iota * 8 for m in range(M)]

            def d_body(d, accs):
                dcol = jnp.full((16,), d, dtype=jnp.int32)
                cv = plsc.load_gather(crows_v, [bidx, dcol])
                return tuple(
                    accs[m] + cv * plsc.load_gather(xrows_v, [rowv[m], dcol])
                    for m in range(M)
                )

            accs = lax.fori_loop(
                0, D, d_body,
                tuple(jnp.zeros((16,), jnp.float32) for _ in range(M)))
            for m in range(M):
                plsc.store_scatter(pos_v, [rowv[m]], accs[m])
            return carry2

        lax.fori_loop(0, GROUPS, group_body, 0)
        pltpu.sync_copy(pos_v, pos_hbm.at[pl.ds(base * M, CHUNK * M)])
        pltpu.sync_copy(crows_v, cemb_hbm.at[pl.ds(base, CHUNK)])
        return carry

    lax.fori_loop(0, NCHUNKS, chunk_body, 0)


_BLK = 2048


def _tc_loss(pos_ref, cen_ref, neg_ref, out_ref):
    i = pl.program_id(0)
    pos = pos_ref[...]
    cen = cen_ref[...]
    neg = neg_ref[...]
    negs = lax.dot_general(cen, neg, (((1,), (1,)), ((), ())),
                           preferred_element_type=jnp.float32)
    nmax = jnp.max(negs, axis=1, keepdims=True)
    s = jnp.sum(jnp.exp(negs - nmax), axis=1, keepdims=True)
    a = jnp.maximum(pos, nmax)
    r = a + jnp.log(jnp.exp(pos - a) + jnp.exp(nmax - a) * s) - pos
    part = jnp.sum(r) * (1.0 / (NB * M))

    @pl.when(i == 0)
    def _():
        out_ref[0, 0] = 0.0

    out_ref[0, 0] += part


_tc_call = pl.pallas_call(
    _tc_loss,
    grid=(NB // _BLK,),
    in_specs=[
        pl.BlockSpec((_BLK, M), lambda i: (i, 0)),
        pl.BlockSpec((_BLK, D), lambda i: (i, 0)),
        pl.BlockSpec((K, D), lambda i: (0, 0)),
    ],
    out_specs=pl.BlockSpec(memory_space=pltpu.SMEM),
    out_shape=jax.ShapeDtypeStruct((1, 1), jnp.float32),
)


def kernel(center_words, context_words, embedding, neg_labels):
    cen = center_words.astype(jnp.int32)
    ctx2d = context_words.astype(jnp.int32).reshape(NB * M)
    pos_flat, cemb, nemb = _sc_scores(cen, ctx2d, embedding,
                                      neg_labels.astype(jnp.int32))
    pos = pos_flat.reshape(NB, M)
    loss = _tc_call(pos, cemb, nemb)
    return loss[0, 0]
